# Initial kernel scaffold; baseline (speedup 1.0000x reference)
import jax
import jax.numpy as jnp
from jax.experimental import pallas as pl


def kernel(feature_map, roi_bboxes):
    raise NotImplementedError("write your pallas kernel here")


# speedup vs baseline: 5.7529x; 5.7529x over previous
import functools

import jax
import jax.numpy as jnp
from jax import lax
from jax.experimental import pallas as pl
from jax.experimental.pallas import tpu as pltpu
from jax.experimental.pallas import tpu_sc as plsc

_POOL = 7
_H = 32
_W = 32
_C = 512
_K = 30


def _roi_pool_tc_kernel(bbox_ref, fm_ref, out_ref):
    b = pl.program_id(0)
    j = pl.program_id(1)
    n_steps = pl.num_programs(1)
    r0 = (b * n_steps + j) * _K
    boxes = bbox_ref[pl.ds(r0, _K), :]

    frac = (lax.broadcasted_iota(jnp.int32, (1, _POOL), 1)
            .astype(jnp.float32) / (_POOL - 1))

    def make_w(c1, c2, dim):
        coords = c1 * (dim - 1) + frac * ((c2 - c1) * (dim - 1))
        c0f = jnp.floor(coords)
        w = coords - c0f
        c0i = c0f.astype(jnp.int32)
        lo = jnp.clip(c0i, 0, dim - 1)
        hi = jnp.clip(c0i + 1, 0, dim - 1)
        valid = (coords >= 0.0) & (coords <= dim - 1)
        wa = jnp.where(valid, 1.0 - w, 0.0)
        wb = jnp.where(valid, w, 0.0)
        iota = lax.broadcasted_iota(jnp.int32, (_K, _POOL, dim), 2)
        return (jnp.where(iota == lo[:, :, None], wa[:, :, None], 0.0)
                + jnp.where(iota == hi[:, :, None], wb[:, :, None], 0.0))

    wy = make_w(boxes[:, 0:1], boxes[:, 2:3], _H)
    wx = make_w(boxes[:, 1:2], boxes[:, 3:4], _W)

    fmt = fm_ref[0]
    s = lax.dot_general(
        wx.reshape(_K * _POOL, _W), fmt,
        dimension_numbers=(((1,), (0,)), ((), ())),
        preferred_element_type=jnp.float32,
    ).reshape(_K, _POOL, _H, _C)
    out_ref[...] = lax.dot_general(
        wy, s,
        dimension_numbers=(((2,), (2,)), ((0,), (0,))),
        preferred_element_type=jnp.float32,
    )


def _roi_pool_tc(feature_map, roi_bboxes):
    B, N = roi_bboxes.shape[0], roi_bboxes.shape[1]
    boxes = roi_bboxes.reshape(B * N, 4)
    fmt = jnp.swapaxes(feature_map, 1, 2)
    out = pl.pallas_call(
        _roi_pool_tc_kernel,
        grid=(B, N // _K),
        in_specs=[
            pl.BlockSpec((B * N, 4), lambda b, j: (0, 0)),
            pl.BlockSpec((1, _W, _H, _C), lambda b, j: (b, 0, 0, 0)),
        ],
        out_specs=pl.BlockSpec(
            (_K, _POOL, _POOL, _C), lambda b, j: (b * (N // _K) + j, 0, 0, 0)),
        out_shape=jax.ShapeDtypeStruct((B * N, _POOL, _POOL, _C), jnp.float32),
    )(boxes, fmt)
    return out.reshape(B, N, _POOL, _POOL, _C)


def _roi_pool_tc2_kernel(bbox_ref, fm_ref, r7y_ref, r7x_ref, ey_ref, ex_ref,
                         out_ref):
    b = pl.program_id(0)
    j = pl.program_id(1)
    n_steps = pl.num_programs(1)
    r0 = (b * n_steps + j) * _K
    boxes = bbox_ref[pl.ds(r0, _K), :]

    frac = (lax.broadcasted_iota(jnp.int32, (1, _POOL), 1)
            .astype(jnp.float32) / (_POOL - 1))

    def make_w(c1, c2, dim):
        coords = c1 * (dim - 1) + frac * ((c2 - c1) * (dim - 1))
        c0f = jnp.floor(coords)
        w = coords - c0f
        c0i = c0f.astype(jnp.int32)
        lo = jnp.clip(c0i, 0, dim - 1)
        hi = jnp.clip(c0i + 1, 0, dim - 1)
        valid = (coords >= 0.0) & (coords <= dim - 1)
        wa = jnp.where(valid, 1.0 - w, 0.0)
        wb = jnp.where(valid, w, 0.0)
        iota = lax.broadcasted_iota(jnp.int32, (_K, _POOL, dim), 2)
        return (jnp.where(iota == lo[:, :, None], wa[:, :, None], 0.0)
                + jnp.where(iota == hi[:, :, None], wb[:, :, None], 0.0))

    wy = make_w(boxes[:, 0:1], boxes[:, 2:3], _H).reshape(_K * _POOL, _H)
    wx = make_w(boxes[:, 1:2], boxes[:, 3:4], _W).reshape(_K * _POOL, _W)

    dn = (((1,), (0,)), ((), ()))
    ty = lax.dot_general(wy.astype(jnp.bfloat16), ey_ref[...],
                         dimension_numbers=dn,
                         preferred_element_type=jnp.float32)
    yext = lax.dot_general(r7y_ref[...], ty.astype(jnp.bfloat16),
                           dimension_numbers=dn,
                           preferred_element_type=jnp.float32)
    tx = lax.dot_general(wx.astype(jnp.bfloat16), ex_ref[...],
                         dimension_numbers=dn,
                         preferred_element_type=jnp.float32)
    xext = lax.dot_general(r7x_ref[...], tx.astype(jnp.bfloat16),
                           dimension_numbers=dn,
                           preferred_element_type=jnp.float32)
    w2 = (yext * xext).astype(jnp.bfloat16)
    out_ref[0, 0] = lax.dot_general(
        w2, fm_ref[0],
        dimension_numbers=dn, preferred_element_type=jnp.float32)


def _roi_pool_tc2(feature_map, roi_bboxes):
    B, N = roi_bboxes.shape[0], roi_bboxes.shape[1]
    nj = N // _K
    boxes = roi_bboxes.reshape(B * N, 4)
    fm16 = feature_map.reshape(B, _H * _W, _C).astype(jnp.bfloat16)
    rows = jnp.arange(_K * 49)
    r_idx = rows // 49
    q_idx = (rows // _POOL) % _POOL
    p_idx = rows % _POOL
    cols70 = jnp.arange(_K * _POOL)
    r7y = (cols70[None, :] == r_idx[:, None] * _POOL + q_idx[:, None]
           ).astype(jnp.bfloat16)
    r7x = (cols70[None, :] == r_idx[:, None] * _POOL + p_idx[:, None]
           ).astype(jnp.bfloat16)
    colj = jnp.arange(_H * _W)
    ey = (jnp.arange(_H)[:, None] == colj[None, :] // _W).astype(jnp.bfloat16)
    ex = (jnp.arange(_W)[:, None] == colj[None, :] % _W).astype(jnp.bfloat16)
    out = pl.pallas_call(
        _roi_pool_tc2_kernel,
        grid=(B, nj),
        in_specs=[
            pl.BlockSpec((B * N, 4), lambda b, j: (0, 0)),
            pl.BlockSpec((1, _H * _W, _C), lambda b, j: (b, 0, 0)),
            pl.BlockSpec((_K * 49, _K * _POOL), lambda b, j: (0, 0)),
            pl.BlockSpec((_K * 49, _K * _POOL), lambda b, j: (0, 0)),
            pl.BlockSpec((_H, _H * _W), lambda b, j: (0, 0)),
            pl.BlockSpec((_W, _H * _W), lambda b, j: (0, 0)),
        ],
        out_specs=pl.BlockSpec(
            (1, 1, _K * 49, _C), lambda b, j: (b, j, 0, 0)),
        out_shape=jax.ShapeDtypeStruct((B, nj, _K * 49, _C), jnp.float32),
    )(boxes, fm16, r7y, r7x, ey, ex)
    return out.reshape(B, N, _POOL, _POOL, _C)


_NC = 2
_NS = 16
_NW = _NC * _NS
_L = 16
_PHASES = ((0, 28), (28, 21))
_GROWS = 112


def _splat(ref, i):
    return plsc.load_gather(ref, [jnp.full((_L,), i, jnp.int32)])


def _sc_body(nroi, rpw, npb, fm_hbm, bbox_hbm, out_hbm,
             bbox_v, coord_v, wprod_v, g_v, out_v, sem, osem):
    wid = lax.axis_index("s") * _NC + lax.axis_index("c")
    r_base = wid * rpw
    pltpu.sync_copy(bbox_hbm.at[pl.ds(r_base * 4, rpw * 4)],
                    bbox_v.at[pl.ds(0, rpw * 4)])

    lanes = lax.iota(jnp.int32, _L)
    lanes_f = lanes.astype(jnp.float32)
    frac = lanes_f * (1.0 / (_POOL - 1))

    def per_roi(t, carry):
        r = r_base + t

        @pl.when(r < nroi)
        def _():
            y1 = _splat(bbox_v, 4 * t + 0)
            x1 = _splat(bbox_v, 4 * t + 1)
            y2 = _splat(bbox_v, 4 * t + 2)
            x2 = _splat(bbox_v, 4 * t + 3)
            b = r // npb

            ys = y1 * (_H - 1.0) + frac * ((y2 - y1) * (_H - 1.0))
            xs = x1 * (_W - 1.0) + frac * ((x2 - x1) * (_W - 1.0))
            y0i = ys.astype(jnp.int32)
            x0i = xs.astype(jnp.int32)
            wyv = ys - y0i.astype(jnp.float32)
            wxv = xs - x0i.astype(jnp.float32)
            y0c = jnp.clip(y0i, 0, _H - 1)
            x0c = jnp.clip(x0i, 0, _W - 1)
            y1c = jnp.clip(y0i + 1, 0, _H - 1)
            x1c = jnp.clip(x0i + 1, 0, _W - 1)

            coord_v[pl.ds(0, _L)] = y0c.astype(jnp.float32)
            coord_v[pl.ds(_L, _L)] = y1c.astype(jnp.float32)
            coord_v[pl.ds(2 * _L, _L)] = x0c.astype(jnp.float32)
            coord_v[pl.ds(3 * _L, _L)] = x1c.astype(jnp.float32)
            coord_v[pl.ds(4 * _L, _L)] = 1.0 - wyv
            coord_v[pl.ds(5 * _L, _L)] = wyv
            coord_v[pl.ds(6 * _L, _L)] = 1.0 - wxv
            coord_v[pl.ds(7 * _L, _L)] = wxv

            for j in range(4):
                kv = lanes + j * _L
                qv = kv // _POOL
                pv = kv - qv * _POOL
                wya = plsc.load_gather(coord_v, [qv + 4 * _L])
                wyb = plsc.load_gather(coord_v, [qv + 5 * _L])
                wxa = plsc.load_gather(coord_v, [pv + 6 * _L])
                wxb = plsc.load_gather(coord_v, [pv + 7 * _L])
                wprod_v[pl.ds(j * _L, _L)] = wya * wxa
                wprod_v[pl.ds(64 + j * _L, _L)] = wya * wxb
                wprod_v[pl.ds(128 + j * _L, _L)] = wyb * wxa
                wprod_v[pl.ds(192 + j * _L, _L)] = wyb * wxb

            for k0, npix in _PHASES:
                ndma = (npix * 4 + _L - 1) // _L
                cps = []
                for j in range(ndma):
                    pos = lanes + j * _L + k0 * 4
                    kv = pos // 4
                    cor = pos - kv * 4
                    qv = kv // _POOL
                    pv = kv - qv * _POOL
                    y0g = plsc.load_gather(coord_v, [qv])
                    y1g = plsc.load_gather(coord_v, [qv + _L])
                    x0g = plsc.load_gather(coord_v, [pv + 2 * _L])
                    x1g = plsc.load_gather(coord_v, [pv + 3 * _L])
                    yv = jnp.where(cor >= 2, y1g, y0g).astype(jnp.int32)
                    xv = jnp.where((cor & 1) == 1, x1g, x0g).astype(jnp.int32)
                    idx = jnp.clip(b * (_H * _W) + yv * _W + xv,
                                   0, 4 * _H * _W - 1)
                    cps.append(pltpu.async_copy(
                        fm_hbm.at[idx], g_v.at[pl.ds(j * _L, _L)], sem))
                for cp in cps:
                    cp.wait()

                def per_pixel(k, c2):
                    s00 = _splat(wprod_v, k)
                    s01 = _splat(wprod_v, k + 64)
                    s10 = _splat(wprod_v, k + 128)
                    s11 = _splat(wprod_v, k + 192)
                    gr = (k - k0) * 4
                    for c in range(_C // _L):
                        sl = pl.ds(c * _L, _L)
                        acc = (s00 * g_v[gr, sl] + s01 * g_v[gr + 1, sl]
                               + s10 * g_v[gr + 2, sl] + s11 * g_v[gr + 3, sl])
                        out_v[k, sl] = acc
                    return c2

                lax.fori_loop(k0, k0 + npix, per_pixel, 0, unroll=False)

            pltpu.async_copy(out_v, out_hbm.at[r], osem).wait()
        return carry

    lax.fori_loop(0, rpw, per_roi, 0, unroll=False)


def _roi_pool_sc(feature_map, roi_bboxes):
    B, N = roi_bboxes.shape[0], roi_bboxes.shape[1]
    nroi = B * N
    rpw = (nroi + _NW - 1) // _NW
    fm = feature_map.reshape(B * _H * _W, _C)
    bb = roi_bboxes.reshape(nroi, 4)
    pad = _NW * rpw - nroi
    bbp = jnp.pad(bb, ((0, pad), (0, 0))).reshape(-1)
    body = functools.partial(_sc_body, nroi, rpw, N)
    out = pl.kernel(
        body,
        out_type=jax.ShapeDtypeStruct((nroi, 49, _C), jnp.float32),
        mesh=plsc.VectorSubcoreMesh(
            core_axis_name="c", subcore_axis_name="s",
            num_cores=_NC, num_subcores=_NS),
        scratch_types=[
            pltpu.VMEM((((rpw * 4 + 127) // 128) * 128,), jnp.float32),
            pltpu.VMEM((8 * _L,), jnp.float32),
            pltpu.VMEM((256,), jnp.float32),
            pltpu.VMEM((_GROWS, _C), jnp.float32),
            pltpu.VMEM((49, _C), jnp.float32),
            pltpu.SemaphoreType.DMA,
            pltpu.SemaphoreType.DMA,
        ],
        compiler_params=pltpu.CompilerParams(needs_layout_passes=False),
    )(fm, bbp)
    return out.reshape(B, N, _POOL, _POOL, _C)


_N_TC = 240


def _roi_pool_hybrid(feature_map, roi_bboxes):
    N = roi_bboxes.shape[1]
    out_tc = _roi_pool_tc2(feature_map, roi_bboxes[:, :_N_TC])
    out_sc = _roi_pool_sc(feature_map, roi_bboxes[:, _N_TC:])
    return jnp.concatenate([out_tc, out_sc], axis=1)


def kernel(feature_map, roi_bboxes):
    return _roi_pool_tc2(feature_map, roi_bboxes)
